# Initial kernel scaffold; baseline (speedup 1.0000x reference)
import jax
import jax.numpy as jnp
from jax.experimental import pallas as pl


def kernel(x, table):
    raise NotImplementedError("write your pallas kernel here")


# speedup vs baseline: 4.7524x; 4.7524x over previous
import jax
import jax.numpy as jnp
from jax.experimental import pallas as pl


def kernel(x, table):
    B = x.shape[0]
    M, D = table.shape
    R = 256

    def body(t_ref, o_ref):
        o_ref[...] = jnp.broadcast_to(t_ref[...][None], (B, R, D))

    return pl.pallas_call(
        body,
        grid=(M // R,),
        in_specs=[pl.BlockSpec((R, D), lambda i: (i, 0))],
        out_specs=pl.BlockSpec((B, R, D), lambda i: (0, i, 0)),
        out_shape=jax.ShapeDtypeStruct((B, M, D), table.dtype),
    )(table)


# speedup vs baseline: 5.0442x; 1.0614x over previous
import jax
import jax.numpy as jnp
from jax.experimental import pallas as pl


def kernel(x, table):
    B = x.shape[0]
    M, D = table.shape
    R = 512

    def body(t_ref, o_ref):
        o_ref[...] = jnp.broadcast_to(t_ref[...][None], (B, R, D))

    return pl.pallas_call(
        body,
        grid=(M // R,),
        in_specs=[pl.BlockSpec((R, D), lambda i: (i, 0))],
        out_specs=pl.BlockSpec((B, R, D), lambda i: (0, i, 0)),
        out_shape=jax.ShapeDtypeStruct((B, M, D), table.dtype),
    )(table)


# speedup vs baseline: 5.1827x; 1.0275x over previous
import jax
import jax.numpy as jnp
from jax.experimental import pallas as pl


def kernel(x, table):
    B = x.shape[0]
    M, D = table.shape
    R = 1024

    def body(t_ref, o_ref):
        o_ref[...] = jnp.broadcast_to(t_ref[...][None], (B, R, D))

    return pl.pallas_call(
        body,
        grid=(M // R,),
        in_specs=[pl.BlockSpec((R, D), lambda i: (i, 0))],
        out_specs=pl.BlockSpec((B, R, D), lambda i: (0, i, 0)),
        out_shape=jax.ShapeDtypeStruct((B, M, D), table.dtype),
    )(table)
